# baseline (speedup 1.0000x reference)
import functools

import jax
import jax.numpy as jnp
from jax import lax
from jax.experimental import pallas as pl
from jax.experimental.pallas import tpu as pltpu
from jax.experimental.pallas import tpu_sc as plsc

D_MODEL = 128


@functools.lru_cache(maxsize=None)
def _make_gather(B, V, D):
    info = plsc.get_sparse_core_info()
    NC, NS = info.num_cores, info.num_subcores
    NW = NC * NS
    assert B % NW == 0
    b_per_w = B // NW
    chunk = 200
    depth = 4
    assert b_per_w % chunk == 0 and chunk % 8 == 0
    n_chunks = b_per_w // chunk
    assert n_chunks % depth == 0 and n_chunks >= 2 * depth

    mesh = plsc.VectorSubcoreMesh(core_axis_name="c", subcore_axis_name="s")

    @functools.partial(
        pl.kernel,
        mesh=mesh,
        out_type=jax.ShapeDtypeStruct((B, D), jnp.float32),
        scratch_types=[
            pltpu.VMEM((b_per_w,), jnp.int32),
        ]
        + [pltpu.VMEM((chunk, D), jnp.float32) for _ in range(depth)]
        + [pltpu.SemaphoreType.DMA for _ in range(depth)],
    )
    def gather_kernel(idx_hbm, table_hbm, out_hbm, idx_v, *rest):
        bufs = rest[:depth]
        sems = rest[depth:]
        wid = lax.axis_index("s") * NC + lax.axis_index("c")
        base = wid * b_per_w
        pltpu.sync_copy(idx_hbm.at[pl.ds(base, b_per_w)], idx_v)

        def start(c, b):
            pltpu.async_copy(
                table_hbm.at[idx_v.at[pl.ds(c * chunk, chunk)]], bufs[b], sems[b]
            )

        def finish(c, b):
            pltpu.make_async_copy(
                table_hbm.at[idx_v.at[pl.ds(c * chunk, chunk)]], bufs[b], sems[b]
            ).wait()
            pltpu.sync_copy(bufs[b], out_hbm.at[pl.ds(base + c * chunk, chunk)])

        for b in range(depth):
            start(b, b)

        def body(g, _):
            for b in range(depth):
                finish(g + b, b)
                start(g + b + depth, b)
            return 0

        lax.fori_loop(0, (n_chunks - depth) // depth, lambda i, c: body(i * depth, c), 0)
        for b in range(depth):
            finish(n_chunks - depth + b, b)

    return gather_kernel


@functools.lru_cache(maxsize=None)
def _make_place(NB, S, D, nbk, k, aliased):
    CB = 16
    assert nbk % CB == 0
    grid = nbk // CB

    def body(*refs):
        in_ref, out_ref = refs[-2], refs[-1]
        for j in range(CB):
            out_ref[j] = in_ref[pl.ds(j * S, S)]

    in_specs = [pl.BlockSpec((CB * S, D), lambda i: (i, 0))]
    if aliased:
        in_specs = [pl.BlockSpec(memory_space=pl.ANY)] + in_specs
    return pl.pallas_call(
        body,
        grid=(grid,),
        in_specs=in_specs,
        out_specs=pl.BlockSpec((CB, S, D), lambda i, k=k: (k * grid + i, 0, 0)),
        out_shape=jax.ShapeDtypeStruct((NB, S, D), jnp.float32),
        input_output_aliases={0: 0} if aliased else {},
    )


def kernel(tokens, masks, table, pe):
    NB, S = tokens.shape
    V, D = table.shape
    idx = tokens.reshape(-1).astype(jnp.int32)
    K = 4
    nbk = NB // K
    gather = _make_gather(nbk * S, V, D)
    x = None
    for k in range(K):
        piece = gather(lax.slice(idx, (k * nbk * S,), ((k + 1) * nbk * S,)), table)
        place = _make_place(NB, S, D, nbk, k, aliased=k > 0)
        x = place(piece) if k == 0 else place(x, piece)
    pos_embed = pe[:S][None, :, :]
    return (x, masks, pos_embed)


# speedup vs baseline: 1.9084x; 1.9084x over previous
import functools

import jax
import jax.numpy as jnp
from jax import lax
from jax.experimental import pallas as pl
from jax.experimental.pallas import tpu as pltpu
from jax.experimental.pallas import tpu_sc as plsc

D_MODEL = 128


@functools.lru_cache(maxsize=None)
def _make_gather(NB, S, V, D):
    info = plsc.get_sparse_core_info()
    NC, NS = info.num_cores, info.num_subcores
    NW = NC * NS
    B = NB * S
    assert B % NW == 0
    b_per_w = B // NW
    cb = 8
    chunk = cb * S
    depth = 2
    assert b_per_w % chunk == 0 and chunk % 8 == 0
    n_chunks = b_per_w // chunk
    nb_per_w = b_per_w // S
    assert n_chunks % depth == 0 and n_chunks >= 2 * depth

    mesh = plsc.VectorSubcoreMesh(core_axis_name="c", subcore_axis_name="s")

    @functools.partial(
        pl.kernel,
        mesh=mesh,
        out_type=jax.ShapeDtypeStruct((NB, S, D), jnp.float32),
        scratch_types=[
            pltpu.VMEM((b_per_w,), jnp.int32),
        ]
        + [pltpu.VMEM((chunk, D), jnp.float32) for _ in range(depth)]
        + [pltpu.SemaphoreType.DMA for _ in range(depth)],
    )
    def gather_kernel(idx_hbm, table_hbm, out_hbm, idx_v, *rest):
        bufs = rest[:depth]
        sems = rest[depth:]
        wid = lax.axis_index("s") * NC + lax.axis_index("c")
        base = wid * b_per_w
        nb_base = wid * nb_per_w
        pltpu.sync_copy(idx_hbm.at[pl.ds(base, b_per_w)], idx_v)

        def start(c, b):
            pltpu.async_copy(
                table_hbm.at[idx_v.at[pl.ds(c * chunk, chunk)]], bufs[b], sems[b]
            )

        def finish(c, b):
            pltpu.make_async_copy(
                table_hbm.at[idx_v.at[pl.ds(c * chunk, chunk)]], bufs[b], sems[b]
            ).wait()
            for j in range(cb):
                pltpu.sync_copy(
                    bufs[b].at[pl.ds(j * S, S)], out_hbm.at[nb_base + c * cb + j]
                )

        for b in range(depth):
            start(b, b)

        def body(g, _):
            for b in range(depth):
                finish(g + b, b)
                start(g + b + depth, b)
            return 0

        lax.fori_loop(0, (n_chunks - depth) // depth, lambda i, c: body(i * depth, c), 0)
        for b in range(depth):
            finish(n_chunks - depth + b, b)

    return gather_kernel


def kernel(tokens, masks, table, pe):
    NB, S = tokens.shape
    idx = tokens.reshape(-1).astype(jnp.int32)
    gather = _make_gather(NB, S, table.shape[0], table.shape[1])
    x = gather(idx, table)
    pos_embed = pe[:S][None, :, :]
    return (x, masks, pos_embed)
